# baseline (speedup 1.0000x reference)
import functools

import jax
import jax.numpy as jnp
from jax import lax
from jax.experimental import pallas as pl
from jax.experimental.pallas import tpu as pltpu
from jax.experimental.pallas import tpu_sc as plsc

N_NODES = 10000
N_EDGES = 320000
F = 128

NC = 2
NS = 16
L = 16
NW = NC * NS
EPW = N_EDGES // NW
CH_B = 200
CH_D = 1280
FG = 16
RPG = F // FG

_DN_CONTRACT_MINOR = (((1,), (1,)), ((), ()))


def _node_tables_body(x_ref, wl_ref, w1_ref, cd_ref, cs_ref):
    x = x_ref[...]
    wl = wl_ref[...]
    a = wl[:, :F]
    b = wl[:, F:]
    w1 = w1_ref[...]
    cd0 = lax.dot_general(x, a - b, _DN_CONTRACT_MINOR,
                          preferred_element_type=jnp.float32)
    cs0 = lax.dot_general(x, b, _DN_CONTRACT_MINOR,
                          preferred_element_type=jnp.float32)
    cd_ref[...] = lax.dot_general(cd0, w1, _DN_CONTRACT_MINOR,
                                  preferred_element_type=jnp.float32)
    cs_ref[...] = lax.dot_general(cs0, w1, _DN_CONTRACT_MINOR,
                                  preferred_element_type=jnp.float32)


def _node_tables(x, w_label, w1):
    nb = 2000
    grid = (N_NODES // nb,)
    return pl.pallas_call(
        _node_tables_body,
        grid=grid,
        in_specs=[
            pl.BlockSpec((nb, F), lambda i: (i, 0)),
            pl.BlockSpec((F, 2 * F), lambda i: (0, 0)),
            pl.BlockSpec((F, F), lambda i: (0, 0)),
        ],
        out_specs=[
            pl.BlockSpec((nb, F), lambda i: (i, 0)),
            pl.BlockSpec((nb, F), lambda i: (i, 0)),
        ],
        out_shape=[
            jax.ShapeDtypeStruct((N_NODES, F), jnp.float32),
            jax.ShapeDtypeStruct((N_NODES, F), jnp.float32),
        ],
    )(x, w_label, w1)


NCH_B = EPW // CH_B


def _gather_add_body(cd_hbm, cs_hbm, src_hbm, dst_hbm, g_hbm,
                     didx_v, sidx_v, cdr_v, csr_v,
                     semi, semg0, semg1, semw0, semw1):
    semg = (semg0, semg1)
    semw = (semw0, semw1)
    wid = lax.axis_index("s") * NC + lax.axis_index("c")
    base_w = wid * EPW

    cpi0 = pltpu.async_copy(dst_hbm.at[pl.ds(base_w, EPW)], didx_v, semi)
    cpi1 = pltpu.async_copy(src_hbm.at[pl.ds(base_w, EPW)], sidx_v, semi)
    cpi0.wait()
    cpi1.wait()

    def start_gather(c, b):
        sl = pl.ds(c * CH_B, CH_B)
        pltpu.async_copy(cd_hbm.at[didx_v.at[sl]], cdr_v.at[b], semg[b])
        pltpu.async_copy(cs_hbm.at[sidx_v.at[sl]], csr_v.at[b], semg[b])

    def wait_gather(c, b):
        sl = pl.ds(c * CH_B, CH_B)
        pltpu.make_async_copy(cd_hbm.at[didx_v.at[sl]], cdr_v.at[b],
                              semg[b]).wait()
        pltpu.make_async_copy(cs_hbm.at[sidx_v.at[sl]], csr_v.at[b],
                              semg[b]).wait()

    def wait_write(c, b):
        pltpu.make_async_copy(cdr_v.at[b],
                              g_hbm.at[pl.ds(base_w + c * CH_B, CH_B), :],
                              semw[b]).wait()

    start_gather(0, 0)

    def process(c, b, first, last):
        b2 = 1 - b
        wait_gather(c, b)
        if not first:
            wait_write(c - 1, b2)
        if not last:
            start_gather(c + 1, b2)

        def row(e, c2):
            for j in range(F // L):
                s = pl.ds(j * L, L)
                cdr_v[b, e, s] = cdr_v[b, e, s] + csr_v[b, e, s]
            return c2

        lax.fori_loop(0, CH_B, row, 0)
        pltpu.async_copy(cdr_v.at[b],
                         g_hbm.at[pl.ds(base_w + c * CH_B, CH_B), :], semw[b])

    process(0, 0, True, False)

    def pair(ci, carry):
        c = 1 + 2 * ci
        process(c, 1, False, False)
        process(c + 1, 0, False, False)
        return carry

    lax.fori_loop(0, (NCH_B - 2) // 2, pair, 0)
    process(NCH_B - 1, 1, False, True)
    wait_write(NCH_B - 1, 1)


def _gather_add(cd, cs, src, dst):
    mesh = plsc.VectorSubcoreMesh(
        core_axis_name="c", subcore_axis_name="s",
        num_cores=NC, num_subcores=NS)
    fn = pl.kernel(
        _gather_add_body,
        out_type=jax.ShapeDtypeStruct((N_EDGES, F), jnp.float32),
        mesh=mesh,
        compiler_params=pltpu.CompilerParams(needs_layout_passes=False),
        scratch_types=[
            pltpu.VMEM((EPW,), jnp.int32),
            pltpu.VMEM((EPW,), jnp.int32),
            pltpu.VMEM((2, CH_B, F), jnp.float32),
            pltpu.VMEM((2, CH_B, F), jnp.float32),
            pltpu.SemaphoreType.DMA,
            pltpu.SemaphoreType.DMA,
            pltpu.SemaphoreType.DMA,
            pltpu.SemaphoreType.DMA,
            pltpu.SemaphoreType.DMA,
        ],
    )
    return fn(cd, cs, src, dst)


def _edge_mlp_body(g_ref, xl_ref, w1_ref, b1_ref, w2_ref, b2_ref, mt_ref):
    t = lax.dot_general(xl_ref[...], w1_ref[...], _DN_CONTRACT_MINOR,
                        preferred_element_type=jnp.float32)
    h = jnp.maximum(g_ref[...] + t + b1_ref[...], 0.0)
    mt = lax.dot_general(w2_ref[...], h, _DN_CONTRACT_MINOR,
                         preferred_element_type=jnp.float32)
    mt_ref[...] = (mt + b2_ref[...]).reshape(FG, RPG, mt.shape[-1])


def _edge_mlp(g, x_label, w1, b1, w2, b2):
    eb = 2560
    grid = (N_EDGES // eb,)
    return pl.pallas_call(
        _edge_mlp_body,
        grid=grid,
        in_specs=[
            pl.BlockSpec((eb, F), lambda i: (i, 0)),
            pl.BlockSpec((eb, F), lambda i: (i, 0)),
            pl.BlockSpec((F, F), lambda i: (0, 0)),
            pl.BlockSpec((1, F), lambda i: (0, 0)),
            pl.BlockSpec((F, F), lambda i: (0, 0)),
            pl.BlockSpec((F, 1), lambda i: (0, 0)),
        ],
        out_specs=pl.BlockSpec((FG, RPG, eb), lambda i: (0, 0, i)),
        out_shape=jax.ShapeDtypeStruct((FG, RPG, N_EDGES), jnp.float32),
    )(g, x_label, w1, b1, w2, b2)


N_CH_D = N_EDGES // CH_D


def _segmax_body(mt_hbm, dst_hbm, out_hbm, didx_v, mrow_v, acc_v, probe_v,
                 semd0, semd1):
    semd = (semd0, semd1)
    wid = lax.axis_index("s") * NC + lax.axis_index("c")
    fg = wid % FG
    half = wid // FG
    neg_inf = jnp.float32(float("-inf"))
    iota = lax.iota(jnp.int32, L)
    rconst = [jnp.full((L,), r, jnp.int32) for r in range(RPG)]

    def init(i, c):
        for r in range(RPG):
            acc_v[r, pl.ds(i * L, L)] = jnp.full((L,), neg_inf, jnp.float32)
        return c

    lax.fori_loop(0, N_NODES // L, init, 0)

    def start_dma(k, b):
        base = (2 * k + half) * CH_D
        pltpu.async_copy(dst_hbm.at[pl.ds(base, CH_D)], didx_v.at[b], semd[b])
        pltpu.async_copy(mt_hbm.at[fg, :, pl.ds(base, CH_D)], mrow_v.at[b],
                         semd[b])

    def wait_dma(k, b):
        base = (2 * k + half) * CH_D
        pltpu.make_async_copy(dst_hbm.at[pl.ds(base, CH_D)], didx_v.at[b],
                              semd[b]).wait()
        pltpu.make_async_copy(mt_hbm.at[fg, :, pl.ds(base, CH_D)],
                              mrow_v.at[b], semd[b]).wait()

    def update(dstv, vals, mask):
        plsc.store_scatter(probe_v, [dstv], iota, mask=mask)
        got = plsc.load_gather(probe_v, [dstv])
        win = mask & (got == iota)
        for r in range(RPG):
            cur = plsc.load_gather(acc_v, [rconst[r], dstv])
            plsc.store_scatter(acc_v, [rconst[r], dstv],
                               jnp.maximum(cur, vals[r]), mask=win)
        return mask & (~win)

    def process(k, b, last):
        b2 = 1 - b
        wait_dma(k, b)
        if not last:
            start_dma(k + 1, b2)

        ones = jnp.ones((L,), jnp.bool_)

        def vec(i, orv):
            s = pl.ds(i * L, L)
            dstv = didx_v[b, s]
            vals = [mrow_v[b, r, s] for r in range(RPG)]
            pend = update(dstv, vals, ones)
            pend = update(dstv, vals, pend)
            return orv | pend

        orv = lax.fori_loop(0, CH_D // L, vec, jnp.zeros((L,), jnp.bool_))

        @pl.when(jnp.max(orv.astype(jnp.int32)) > 0)
        def _careful():
            def vec2(i, c2):
                s = pl.ds(i * L, L)
                dstv = didx_v[b, s]
                vals = [mrow_v[b, r, s] for r in range(RPG)]
                pend0 = update(dstv, vals, ones)

                def cond(p):
                    return jnp.max(p.astype(jnp.int32)) > 0

                def body(p):
                    return update(dstv, vals, p)

                lax.while_loop(cond, body, pend0)
                return c2

            lax.fori_loop(0, CH_D // L, vec2, 0)

    start_dma(0, 0)
    process(0, 0, False)

    def pair(ci, carry):
        k = 1 + 2 * ci
        process(k, 1, False)
        process(k + 1, 0, False)
        return carry

    lax.fori_loop(0, 61, pair, 0)
    process(123, 1, False)
    process(124, 0, True)
    pltpu.sync_copy(acc_v, out_hbm.at[half, pl.ds(fg * RPG, RPG), :])


def _segment_max(mt, dst):
    mesh = plsc.VectorSubcoreMesh(
        core_axis_name="c", subcore_axis_name="s",
        num_cores=NC, num_subcores=NS)
    fn = pl.kernel(
        _segmax_body,
        out_type=jax.ShapeDtypeStruct((2, F, N_NODES), jnp.float32),
        mesh=mesh,
        compiler_params=pltpu.CompilerParams(needs_layout_passes=False),
        scratch_types=[
            pltpu.VMEM((2, CH_D), jnp.int32),
            pltpu.VMEM((2, RPG, CH_D), jnp.float32),
            pltpu.VMEM((RPG, N_NODES), jnp.float32),
            pltpu.VMEM((N_NODES,), jnp.int32),
            pltpu.SemaphoreType.DMA,
            pltpu.SemaphoreType.DMA,
        ],
    )
    return fn(mt, dst)


def _merge_body(p_ref, out_ref):
    neg_inf = jnp.float32(float("-inf"))
    mx = jnp.maximum(p_ref[0], p_ref[1])
    out_ref[...] = jnp.where(mx == neg_inf, jnp.float32(0.0), mx)


def _merge_halves(p):
    return pl.pallas_call(
        _merge_body,
        grid=(1,),
        in_specs=[pl.BlockSpec((2, F, N_NODES), lambda i: (0, 0, 0))],
        out_specs=pl.BlockSpec((F, N_NODES), lambda i: (0, 0)),
        out_shape=jax.ShapeDtypeStruct((F, N_NODES), jnp.float32),
    )(p)


def kernel(x, edge_index, x_label, W_label, W1, b1, W2, b2):
    src = edge_index[0]
    dst = edge_index[1]
    cd, cs = _node_tables(x, W_label, W1)
    g = _gather_add(cd, cs, src, dst)
    mt = _edge_mlp(g, x_label, W1, b1.reshape(1, F), W2, b2.reshape(F, 1))
    p = _segment_max(mt, dst)
    outt = _merge_halves(p)
    return outt.T


# speedup vs baseline: 1.5968x; 1.5968x over previous
import functools

import jax
import jax.numpy as jnp
from jax import lax
from jax.experimental import pallas as pl
from jax.experimental.pallas import tpu as pltpu
from jax.experimental.pallas import tpu_sc as plsc

N_NODES = 10000
N_EDGES = 320000
F = 128

NC = 2
NS = 16
L = 16
NW = NC * NS
EPW = N_EDGES // NW
CH_B = 200
CH_D = 1280
FG = 16
RPG = F // FG

_DN_CONTRACT_MINOR = (((1,), (1,)), ((), ()))


def _node_tables_body(x_ref, wl_ref, w1_ref, cd_ref, cs_ref):
    x = x_ref[...]
    wl = wl_ref[...]
    a = wl[:, :F]
    b = wl[:, F:]
    w1 = w1_ref[...]
    cd0 = lax.dot_general(x, a - b, _DN_CONTRACT_MINOR,
                          preferred_element_type=jnp.float32)
    cs0 = lax.dot_general(x, b, _DN_CONTRACT_MINOR,
                          preferred_element_type=jnp.float32)
    cd_ref[...] = lax.dot_general(cd0, w1, _DN_CONTRACT_MINOR,
                                  preferred_element_type=jnp.float32)
    cs_ref[...] = lax.dot_general(cs0, w1, _DN_CONTRACT_MINOR,
                                  preferred_element_type=jnp.float32)


def _node_tables(x, w_label, w1):
    nb = 2000
    grid = (N_NODES // nb,)
    return pl.pallas_call(
        _node_tables_body,
        grid=grid,
        in_specs=[
            pl.BlockSpec((nb, F), lambda i: (i, 0)),
            pl.BlockSpec((F, 2 * F), lambda i: (0, 0)),
            pl.BlockSpec((F, F), lambda i: (0, 0)),
        ],
        out_specs=[
            pl.BlockSpec((nb, F), lambda i: (i, 0)),
            pl.BlockSpec((nb, F), lambda i: (i, 0)),
        ],
        out_shape=[
            jax.ShapeDtypeStruct((N_NODES, F), jnp.float32),
            jax.ShapeDtypeStruct((N_NODES, F), jnp.float32),
        ],
    )(x, w_label, w1)


NCH_B = EPW // CH_B


def _gather_add_body(cd_hbm, cs_hbm, src_hbm, dst_hbm, g_hbm,
                     didx_v, sidx_v, cdr_v, csr_v,
                     semi, semg0, semg1, semw0, semw1):
    semg = (semg0, semg1)
    semw = (semw0, semw1)
    wid = lax.axis_index("s") * NC + lax.axis_index("c")
    base_w = wid * EPW

    cpi0 = pltpu.async_copy(dst_hbm.at[pl.ds(base_w, EPW)], didx_v, semi)
    cpi1 = pltpu.async_copy(src_hbm.at[pl.ds(base_w, EPW)], sidx_v, semi)
    cpi0.wait()
    cpi1.wait()

    def start_gather(c, b):
        sl = pl.ds(c * CH_B, CH_B)
        pltpu.async_copy(cd_hbm.at[didx_v.at[sl]], cdr_v.at[b], semg[b])
        pltpu.async_copy(cs_hbm.at[sidx_v.at[sl]], csr_v.at[b], semg[b])

    def wait_gather(c, b):
        sl = pl.ds(c * CH_B, CH_B)
        pltpu.make_async_copy(cd_hbm.at[didx_v.at[sl]], cdr_v.at[b],
                              semg[b]).wait()
        pltpu.make_async_copy(cs_hbm.at[sidx_v.at[sl]], csr_v.at[b],
                              semg[b]).wait()

    def wait_write(c, b):
        pltpu.make_async_copy(cdr_v.at[b],
                              g_hbm.at[pl.ds(base_w + c * CH_B, CH_B), :],
                              semw[b]).wait()

    start_gather(0, 0)

    def process(c, b, first, last):
        b2 = 1 - b
        wait_gather(c, b)
        if not first:
            wait_write(c - 1, b2)
        if not last:
            start_gather(c + 1, b2)

        def row(e, c2):
            for j in range(F // L):
                s = pl.ds(j * L, L)
                cdr_v[b, e, s] = cdr_v[b, e, s] + csr_v[b, e, s]
            return c2

        lax.fori_loop(0, CH_B, row, 0)
        pltpu.async_copy(cdr_v.at[b],
                         g_hbm.at[pl.ds(base_w + c * CH_B, CH_B), :], semw[b])

    process(0, 0, True, False)

    def pair(ci, carry):
        c = 1 + 2 * ci
        process(c, 1, False, False)
        process(c + 1, 0, False, False)
        return carry

    lax.fori_loop(0, (NCH_B - 2) // 2, pair, 0)
    process(NCH_B - 1, 1, False, True)
    wait_write(NCH_B - 1, 1)


def _gather_add(cd, cs, src, dst):
    mesh = plsc.VectorSubcoreMesh(
        core_axis_name="c", subcore_axis_name="s",
        num_cores=NC, num_subcores=NS)
    fn = pl.kernel(
        _gather_add_body,
        out_type=jax.ShapeDtypeStruct((N_EDGES, F), jnp.float32),
        mesh=mesh,
        compiler_params=pltpu.CompilerParams(needs_layout_passes=False),
        scratch_types=[
            pltpu.VMEM((EPW,), jnp.int32),
            pltpu.VMEM((EPW,), jnp.int32),
            pltpu.VMEM((2, CH_B, F), jnp.float32),
            pltpu.VMEM((2, CH_B, F), jnp.float32),
            pltpu.SemaphoreType.DMA,
            pltpu.SemaphoreType.DMA,
            pltpu.SemaphoreType.DMA,
            pltpu.SemaphoreType.DMA,
            pltpu.SemaphoreType.DMA,
        ],
    )
    return fn(cd, cs, src, dst)


def _edge_mlp_body(g_ref, xl_ref, w1_ref, b1_ref, w2_ref, b2_ref, mt_ref):
    t = lax.dot_general(xl_ref[...], w1_ref[...], _DN_CONTRACT_MINOR,
                        preferred_element_type=jnp.float32)
    h = jnp.maximum(g_ref[...] + t + b1_ref[...], 0.0)
    mt = lax.dot_general(w2_ref[...], h, _DN_CONTRACT_MINOR,
                         preferred_element_type=jnp.float32)
    mt_ref[...] = (mt + b2_ref[...]).reshape(FG, RPG, mt.shape[-1])


def _edge_mlp(g, x_label, w1, b1, w2, b2):
    eb = 2560
    grid = (N_EDGES // eb,)
    return pl.pallas_call(
        _edge_mlp_body,
        grid=grid,
        in_specs=[
            pl.BlockSpec((eb, F), lambda i: (i, 0)),
            pl.BlockSpec((eb, F), lambda i: (i, 0)),
            pl.BlockSpec((F, F), lambda i: (0, 0)),
            pl.BlockSpec((1, F), lambda i: (0, 0)),
            pl.BlockSpec((F, F), lambda i: (0, 0)),
            pl.BlockSpec((F, 1), lambda i: (0, 0)),
        ],
        out_specs=pl.BlockSpec((FG, RPG, eb), lambda i: (0, 0, i)),
        out_shape=jax.ShapeDtypeStruct((FG, RPG, N_EDGES), jnp.float32),
    )(g, x_label, w1, b1, w2, b2)


N_CH_D = N_EDGES // CH_D


def _segmax_body(mt_hbm, dst_hbm, out_hbm, didx_v, mrow_v, acc_v, probe_v,
                 semd0, semd1):
    semd = (semd0, semd1)
    wid = lax.axis_index("s") * NC + lax.axis_index("c")
    fg = wid % FG
    half = wid // FG
    neg_inf = jnp.float32(float("-inf"))
    iota = lax.iota(jnp.int32, L)
    rconst = [jnp.full((L,), r, jnp.int32) for r in range(RPG)]

    def init(i, c):
        for r in range(RPG):
            acc_v[r, pl.ds(i * L, L)] = jnp.full((L,), neg_inf, jnp.float32)
        return c

    lax.fori_loop(0, N_NODES // L, init, 0)

    def start_dma(k, b):
        base = (2 * k + half) * CH_D
        pltpu.async_copy(dst_hbm.at[pl.ds(base, CH_D)], didx_v.at[b], semd[b])
        pltpu.async_copy(mt_hbm.at[fg, :, pl.ds(base, CH_D)], mrow_v.at[b],
                         semd[b])

    def wait_dma(k, b):
        base = (2 * k + half) * CH_D
        pltpu.make_async_copy(dst_hbm.at[pl.ds(base, CH_D)], didx_v.at[b],
                              semd[b]).wait()
        pltpu.make_async_copy(mt_hbm.at[fg, :, pl.ds(base, CH_D)],
                              mrow_v.at[b], semd[b]).wait()

    def update(dstv, vals, mask):
        plsc.store_scatter(probe_v, [dstv], iota, mask=mask)
        got = plsc.load_gather(probe_v, [dstv])
        win = mask & (got == iota)
        curs = [plsc.load_gather(acc_v, [rconst[r], dstv])
                for r in range(RPG)]
        news = [jnp.maximum(curs[r], vals[r]) for r in range(RPG)]
        for r in range(RPG):
            plsc.store_scatter(acc_v, [rconst[r], dstv], news[r], mask=win)
        return mask & (~win)

    def process(k, b, last):
        b2 = 1 - b
        wait_dma(k, b)
        if not last:
            start_dma(k + 1, b2)

        ones = jnp.ones((L,), jnp.bool_)

        def vec(i, c2):
            s = pl.ds(i * L, L)
            dstv = didx_v[b, s]
            vals = [mrow_v[b, r, s] for r in range(RPG)]
            pend0 = update(dstv, vals, ones)

            def cond(p):
                return jnp.max(p.astype(jnp.int32)) > 0

            def body(p):
                return update(dstv, vals, p)

            lax.while_loop(cond, body, pend0)
            return c2

        lax.fori_loop(0, CH_D // L, vec, 0)

    start_dma(0, 0)
    process(0, 0, False)

    def pair(ci, carry):
        k = 1 + 2 * ci
        process(k, 1, False)
        process(k + 1, 0, False)
        return carry

    lax.fori_loop(0, 61, pair, 0)
    process(123, 1, False)
    process(124, 0, True)
    pltpu.sync_copy(acc_v, out_hbm.at[half, pl.ds(fg * RPG, RPG), :])


def _segment_max(mt, dst):
    mesh = plsc.VectorSubcoreMesh(
        core_axis_name="c", subcore_axis_name="s",
        num_cores=NC, num_subcores=NS)
    fn = pl.kernel(
        _segmax_body,
        out_type=jax.ShapeDtypeStruct((2, F, N_NODES), jnp.float32),
        mesh=mesh,
        compiler_params=pltpu.CompilerParams(needs_layout_passes=False),
        scratch_types=[
            pltpu.VMEM((2, CH_D), jnp.int32),
            pltpu.VMEM((2, RPG, CH_D), jnp.float32),
            pltpu.VMEM((RPG, N_NODES), jnp.float32),
            pltpu.VMEM((N_NODES,), jnp.int32),
            pltpu.SemaphoreType.DMA,
            pltpu.SemaphoreType.DMA,
        ],
    )
    return fn(mt, dst)


def _merge_body(p_ref, out_ref):
    neg_inf = jnp.float32(float("-inf"))
    mx = jnp.maximum(p_ref[0], p_ref[1])
    out_ref[...] = jnp.where(mx == neg_inf, jnp.float32(0.0), mx)


def _merge_halves(p):
    return pl.pallas_call(
        _merge_body,
        grid=(1,),
        in_specs=[pl.BlockSpec((2, F, N_NODES), lambda i: (0, 0, 0))],
        out_specs=pl.BlockSpec((F, N_NODES), lambda i: (0, 0)),
        out_shape=jax.ShapeDtypeStruct((F, N_NODES), jnp.float32),
    )(p)


def kernel(x, edge_index, x_label, W_label, W1, b1, W2, b2):
    src = edge_index[0]
    dst = edge_index[1]
    cd, cs = _node_tables(x, W_label, W1)
    g = _gather_add(cd, cs, src, dst)
    mt = _edge_mlp(g, x_label, W1, b1.reshape(1, F), W2, b2.reshape(F, 1))
    p = _segment_max(mt, dst)
    outt = _merge_halves(p)
    return outt.T


# speedup vs baseline: 1.7477x; 1.0945x over previous
import functools

import jax
import jax.numpy as jnp
from jax import lax
from jax.experimental import pallas as pl
from jax.experimental.pallas import tpu as pltpu
from jax.experimental.pallas import tpu_sc as plsc

N_NODES = 10000
N_EDGES = 320000
F = 128

NC = 2
NS = 16
L = 16
NW = NC * NS
E_HALF = N_EDGES // 2
EPW = E_HALF // NW
CH_B = 200
NCH_B = EPW // CH_B
CH_D = 640
N_CH_D = E_HALF // CH_D
FG = 16
RPG = F // FG

_DN_CONTRACT_MINOR = (((1,), (1,)), ((), ()))


def _node_tables_body(x_ref, wl_ref, w1_ref, cd_ref, cs_ref):
    x = x_ref[...]
    wl = wl_ref[...]
    a = wl[:, :F]
    b = wl[:, F:]
    w1 = w1_ref[...]
    cd0 = lax.dot_general(x, a - b, _DN_CONTRACT_MINOR,
                          preferred_element_type=jnp.float32)
    cs0 = lax.dot_general(x, b, _DN_CONTRACT_MINOR,
                          preferred_element_type=jnp.float32)
    cd_ref[...] = lax.dot_general(cd0, w1, _DN_CONTRACT_MINOR,
                                  preferred_element_type=jnp.float32)
    cs_ref[...] = lax.dot_general(cs0, w1, _DN_CONTRACT_MINOR,
                                  preferred_element_type=jnp.float32)


def _node_tables(x, w_label, w1):
    nb = 2000
    grid = (N_NODES // nb,)
    return pl.pallas_call(
        _node_tables_body,
        grid=grid,
        in_specs=[
            pl.BlockSpec((nb, F), lambda i: (i, 0)),
            pl.BlockSpec((F, 2 * F), lambda i: (0, 0)),
            pl.BlockSpec((F, F), lambda i: (0, 0)),
        ],
        out_specs=[
            pl.BlockSpec((nb, F), lambda i: (i, 0)),
            pl.BlockSpec((nb, F), lambda i: (i, 0)),
        ],
        out_shape=[
            jax.ShapeDtypeStruct((N_NODES, F), jnp.float32),
            jax.ShapeDtypeStruct((N_NODES, F), jnp.float32),
        ],
    )(x, w_label, w1)


def _gather_add_body(offset, cd_hbm, cs_hbm, src_hbm, dst_hbm, g_hbm,
                     didx_v, sidx_v, cdr_v, csr_v,
                     semi, semg0, semg1, semw0, semw1):
    semg = (semg0, semg1)
    semw = (semw0, semw1)
    wid = lax.axis_index("s") * NC + lax.axis_index("c")
    base_w = offset + wid * EPW
    out_w = wid * EPW

    cpi0 = pltpu.async_copy(dst_hbm.at[pl.ds(base_w, EPW)], didx_v, semi)
    cpi1 = pltpu.async_copy(src_hbm.at[pl.ds(base_w, EPW)], sidx_v, semi)
    cpi0.wait()
    cpi1.wait()

    def start_gather(c, b):
        sl = pl.ds(c * CH_B, CH_B)
        pltpu.async_copy(cd_hbm.at[didx_v.at[sl]], cdr_v.at[b], semg[b])
        pltpu.async_copy(cs_hbm.at[sidx_v.at[sl]], csr_v.at[b], semg[b])

    def wait_gather(c, b):
        sl = pl.ds(c * CH_B, CH_B)
        pltpu.make_async_copy(cd_hbm.at[didx_v.at[sl]], cdr_v.at[b],
                              semg[b]).wait()
        pltpu.make_async_copy(cs_hbm.at[sidx_v.at[sl]], csr_v.at[b],
                              semg[b]).wait()

    def wait_write(c, b):
        pltpu.make_async_copy(cdr_v.at[b],
                              g_hbm.at[pl.ds(out_w + c * CH_B, CH_B), :],
                              semw[b]).wait()

    start_gather(0, 0)

    def process(c, b, first, last):
        b2 = 1 - b
        wait_gather(c, b)
        if not first:
            wait_write(c - 1, b2)
        if not last:
            start_gather(c + 1, b2)

        def row(e, c2):
            for j in range(F // L):
                s = pl.ds(j * L, L)
                cdr_v[b, e, s] = cdr_v[b, e, s] + csr_v[b, e, s]
            return c2

        lax.fori_loop(0, CH_B, row, 0)
        pltpu.async_copy(cdr_v.at[b],
                         g_hbm.at[pl.ds(out_w + c * CH_B, CH_B), :], semw[b])

    process(0, 0, True, False)

    def pair(ci, carry):
        c = 1 + 2 * ci
        process(c, 1, False, False)
        process(c + 1, 0, False, False)
        return carry

    np_ = (NCH_B - 2) // 2
    lax.fori_loop(0, np_, pair, 0)
    for m in range(1 + 2 * np_, NCH_B):
        process(m, m % 2, False, m == NCH_B - 1)
    wait_write(NCH_B - 1, (NCH_B - 1) % 2)


def _gather_add(cd, cs, src, dst, offset):
    mesh = plsc.VectorSubcoreMesh(
        core_axis_name="c", subcore_axis_name="s",
        num_cores=NC, num_subcores=NS)
    fn = pl.kernel(
        functools.partial(_gather_add_body, offset),
        out_type=jax.ShapeDtypeStruct((E_HALF, F), jnp.float32),
        mesh=mesh,
        compiler_params=pltpu.CompilerParams(needs_layout_passes=False),
        scratch_types=[
            pltpu.VMEM((EPW,), jnp.int32),
            pltpu.VMEM((EPW,), jnp.int32),
            pltpu.VMEM((2, CH_B, F), jnp.float32),
            pltpu.VMEM((2, CH_B, F), jnp.float32),
            pltpu.SemaphoreType.DMA,
            pltpu.SemaphoreType.DMA,
            pltpu.SemaphoreType.DMA,
            pltpu.SemaphoreType.DMA,
            pltpu.SemaphoreType.DMA,
        ],
    )
    return fn(cd, cs, src, dst)


def _edge_mlp_body(g_ref, xl_ref, w1_ref, b1_ref, w2_ref, b2_ref, mt_ref):
    t = lax.dot_general(xl_ref[...], w1_ref[...], _DN_CONTRACT_MINOR,
                        preferred_element_type=jnp.float32)
    h = jnp.maximum(g_ref[...] + t + b1_ref[...], 0.0)
    mt = lax.dot_general(w2_ref[...], h, _DN_CONTRACT_MINOR,
                         preferred_element_type=jnp.float32)
    mt_ref[...] = (mt + b2_ref[...]).reshape(FG, RPG, mt.shape[-1])


def _edge_mlp(g, x_label, w1, b1, w2, b2, offset):
    eb = 1280
    grid = (E_HALF // eb,)
    off_b = offset // eb
    return pl.pallas_call(
        _edge_mlp_body,
        grid=grid,
        in_specs=[
            pl.BlockSpec((eb, F), lambda i: (i, 0)),
            pl.BlockSpec((eb, F), lambda i: (off_b + i, 0)),
            pl.BlockSpec((F, F), lambda i: (0, 0)),
            pl.BlockSpec((1, F), lambda i: (0, 0)),
            pl.BlockSpec((F, F), lambda i: (0, 0)),
            pl.BlockSpec((F, 1), lambda i: (0, 0)),
        ],
        out_specs=pl.BlockSpec((FG, RPG, eb), lambda i: (0, 0, i)),
        out_shape=jax.ShapeDtypeStruct((FG, RPG, E_HALF), jnp.float32),
    )(g, x_label, w1, b1, w2, b2)


def _segmax_body(offset, mt_hbm, dst_hbm, out_hbm, didx_v, mrow_v, acc_v,
                 probe_v, semd0, semd1):
    semd = (semd0, semd1)
    wid = lax.axis_index("s") * NC + lax.axis_index("c")
    fg = wid % FG
    half = wid // FG
    neg_inf = jnp.float32(float("-inf"))
    iota = lax.iota(jnp.int32, L)
    rconst = [jnp.full((L,), r, jnp.int32) for r in range(RPG)]

    def init(i, c):
        for r in range(RPG):
            acc_v[r, pl.ds(i * L, L)] = jnp.full((L,), neg_inf, jnp.float32)
        return c

    lax.fori_loop(0, N_NODES // L, init, 0)

    def start_dma(k, b):
        base = (2 * k + half) * CH_D
        pltpu.async_copy(dst_hbm.at[pl.ds(offset + base, CH_D)], didx_v.at[b],
                         semd[b])
        pltpu.async_copy(mt_hbm.at[fg, :, pl.ds(base, CH_D)], mrow_v.at[b],
                         semd[b])

    def wait_dma(k, b):
        base = (2 * k + half) * CH_D
        pltpu.make_async_copy(dst_hbm.at[pl.ds(offset + base, CH_D)],
                              didx_v.at[b], semd[b]).wait()
        pltpu.make_async_copy(mt_hbm.at[fg, :, pl.ds(base, CH_D)],
                              mrow_v.at[b], semd[b]).wait()

    def update(dstv, vals, mask):
        plsc.store_scatter(probe_v, [dstv], iota, mask=mask)
        got = plsc.load_gather(probe_v, [dstv])
        win = mask & (got == iota)
        curs = [plsc.load_gather(acc_v, [rconst[r], dstv])
                for r in range(RPG)]
        news = [jnp.maximum(curs[r], vals[r]) for r in range(RPG)]
        for r in range(RPG):
            plsc.store_scatter(acc_v, [rconst[r], dstv], news[r], mask=win)
        return mask & (~win)

    def process(k, b, last):
        b2 = 1 - b
        wait_dma(k, b)
        if not last:
            start_dma(k + 1, b2)

        ones = jnp.ones((L,), jnp.bool_)

        def vec(i, c2):
            s = pl.ds(i * L, L)
            dstv = didx_v[b, s]
            vals = [mrow_v[b, r, s] for r in range(RPG)]
            pend0 = update(dstv, vals, ones)

            def cond(p):
                return jnp.max(p.astype(jnp.int32)) > 0

            def body(p):
                return update(dstv, vals, p)

            lax.while_loop(cond, body, pend0)
            return c2

        lax.fori_loop(0, CH_D // L, vec, 0)

    start_dma(0, 0)
    process(0, 0, False)

    def pair(ci, carry):
        k = 1 + 2 * ci
        process(k, 1, False)
        process(k + 1, 0, False)
        return carry

    lax.fori_loop(0, 61, pair, 0)
    process(123, 1, False)
    process(124, 0, True)
    pltpu.sync_copy(acc_v, out_hbm.at[half, pl.ds(fg * RPG, RPG), :])


def _segment_max(mt, dst, offset):
    mesh = plsc.VectorSubcoreMesh(
        core_axis_name="c", subcore_axis_name="s",
        num_cores=NC, num_subcores=NS)
    fn = pl.kernel(
        functools.partial(_segmax_body, offset),
        out_type=jax.ShapeDtypeStruct((2, F, N_NODES), jnp.float32),
        mesh=mesh,
        compiler_params=pltpu.CompilerParams(needs_layout_passes=False),
        scratch_types=[
            pltpu.VMEM((2, CH_D), jnp.int32),
            pltpu.VMEM((2, RPG, CH_D), jnp.float32),
            pltpu.VMEM((RPG, N_NODES), jnp.float32),
            pltpu.VMEM((N_NODES,), jnp.int32),
            pltpu.SemaphoreType.DMA,
            pltpu.SemaphoreType.DMA,
        ],
    )
    return fn(mt, dst)


def _merge_body(p1_ref, p2_ref, out_ref):
    neg_inf = jnp.float32(float("-inf"))
    mx = jnp.maximum(jnp.maximum(p1_ref[0], p1_ref[1]),
                     jnp.maximum(p2_ref[0], p2_ref[1]))
    out_ref[...] = jnp.where(mx == neg_inf, jnp.float32(0.0), mx)


def _merge_halves(p1, p2):
    return pl.pallas_call(
        _merge_body,
        grid=(1,),
        in_specs=[
            pl.BlockSpec((2, F, N_NODES), lambda i: (0, 0, 0)),
            pl.BlockSpec((2, F, N_NODES), lambda i: (0, 0, 0)),
        ],
        out_specs=pl.BlockSpec((F, N_NODES), lambda i: (0, 0)),
        out_shape=jax.ShapeDtypeStruct((F, N_NODES), jnp.float32),
    )(p1, p2)


def kernel(x, edge_index, x_label, W_label, W1, b1, W2, b2):
    src = edge_index[0]
    dst = edge_index[1]
    b1r = b1.reshape(1, F)
    b2c = b2.reshape(F, 1)
    cd, cs = _node_tables(x, W_label, W1)
    g1 = _gather_add(cd, cs, src, dst, 0)
    g2 = _gather_add(cd, cs, src, dst, E_HALF)
    mt1 = _edge_mlp(g1, x_label, W1, b1r, W2, b2c, 0)
    mt2 = _edge_mlp(g2, x_label, W1, b1r, W2, b2c, E_HALF)
    p1 = _segment_max(mt1, dst, 0)
    p2 = _segment_max(mt2, dst, E_HALF)
    outt = _merge_halves(p1, p2)
    return outt.T


# speedup vs baseline: 1.8880x; 1.0802x over previous
import functools

import jax
import jax.numpy as jnp
from jax import lax
from jax.experimental import pallas as pl
from jax.experimental.pallas import tpu as pltpu
from jax.experimental.pallas import tpu_sc as plsc

N_NODES = 10000
N_EDGES = 320000
F = 128

NC = 2
NS = 16
L = 16
NW = NC * NS
E_HALF = N_EDGES // 2
EPW = E_HALF // NW
CH_B = 200
NCH_B = EPW // CH_B
CH_D = 640
N_CH_D = E_HALF // CH_D
FG = 16
RPG = F // FG

_DN_CONTRACT_MINOR = (((1,), (1,)), ((), ()))


def _node_tables_body(x_ref, wl_ref, w1_ref, cd_ref, cs_ref):
    x = x_ref[...]
    wl = wl_ref[...]
    a = wl[:, :F]
    b = wl[:, F:]
    w1 = w1_ref[...]
    cd0 = lax.dot_general(x, a - b, _DN_CONTRACT_MINOR,
                          preferred_element_type=jnp.float32)
    cs0 = lax.dot_general(x, b, _DN_CONTRACT_MINOR,
                          preferred_element_type=jnp.float32)
    cd_ref[...] = lax.dot_general(cd0, w1, _DN_CONTRACT_MINOR,
                                  preferred_element_type=jnp.float32)
    cs_ref[...] = lax.dot_general(cs0, w1, _DN_CONTRACT_MINOR,
                                  preferred_element_type=jnp.float32)


def _node_tables(x, w_label, w1):
    nb = 2000
    grid = (N_NODES // nb,)
    return pl.pallas_call(
        _node_tables_body,
        grid=grid,
        in_specs=[
            pl.BlockSpec((nb, F), lambda i: (i, 0)),
            pl.BlockSpec((F, 2 * F), lambda i: (0, 0)),
            pl.BlockSpec((F, F), lambda i: (0, 0)),
        ],
        out_specs=[
            pl.BlockSpec((nb, F), lambda i: (i, 0)),
            pl.BlockSpec((nb, F), lambda i: (i, 0)),
        ],
        out_shape=[
            jax.ShapeDtypeStruct((N_NODES, F), jnp.float32),
            jax.ShapeDtypeStruct((N_NODES, F), jnp.float32),
        ],
    )(x, w_label, w1)


def _gather_add_body(offset, cd_hbm, cs_hbm, src_hbm, dst_hbm, g_hbm,
                     didx_v, sidx_v, cdr_v, csr_v,
                     semi, semg0, semg1, semw0, semw1):
    semg = (semg0, semg1)
    semw = (semw0, semw1)
    wid = lax.axis_index("s") * NC + lax.axis_index("c")
    base_w = offset + wid * EPW
    out_w = wid * EPW

    cpi0 = pltpu.async_copy(dst_hbm.at[pl.ds(base_w, EPW)], didx_v, semi)
    cpi1 = pltpu.async_copy(src_hbm.at[pl.ds(base_w, EPW)], sidx_v, semi)
    cpi0.wait()
    cpi1.wait()

    def start_gather(c, b):
        sl = pl.ds(c * CH_B, CH_B)
        pltpu.async_copy(cd_hbm.at[didx_v.at[sl]], cdr_v.at[b], semg[b])
        pltpu.async_copy(cs_hbm.at[sidx_v.at[sl]], csr_v.at[b], semg[b])

    def wait_gather(c, b):
        sl = pl.ds(c * CH_B, CH_B)
        pltpu.make_async_copy(cd_hbm.at[didx_v.at[sl]], cdr_v.at[b],
                              semg[b]).wait()
        pltpu.make_async_copy(cs_hbm.at[sidx_v.at[sl]], csr_v.at[b],
                              semg[b]).wait()

    def wait_write(c, b):
        pltpu.make_async_copy(cdr_v.at[b],
                              g_hbm.at[pl.ds(out_w + c * CH_B, CH_B), :],
                              semw[b]).wait()

    start_gather(0, 0)

    def process(c, b, first, last):
        b2 = 1 - b
        wait_gather(c, b)
        if not first:
            wait_write(c - 1, b2)
        if not last:
            start_gather(c + 1, b2)

        def row(e, c2):
            for j in range(F // L):
                s = pl.ds(j * L, L)
                cdr_v[b, e, s] = cdr_v[b, e, s] + csr_v[b, e, s]
            return c2

        lax.fori_loop(0, CH_B, row, 0)
        pltpu.async_copy(cdr_v.at[b],
                         g_hbm.at[pl.ds(out_w + c * CH_B, CH_B), :], semw[b])

    process(0, 0, True, False)

    def pair(ci, carry):
        c = 1 + 2 * ci
        process(c, 1, False, False)
        process(c + 1, 0, False, False)
        return carry

    np_ = (NCH_B - 2) // 2
    lax.fori_loop(0, np_, pair, 0)
    for m in range(1 + 2 * np_, NCH_B):
        process(m, m % 2, False, m == NCH_B - 1)
    wait_write(NCH_B - 1, (NCH_B - 1) % 2)


def _gather_add(cd, cs, src, dst, offset):
    mesh = plsc.VectorSubcoreMesh(
        core_axis_name="c", subcore_axis_name="s",
        num_cores=NC, num_subcores=NS)
    fn = pl.kernel(
        functools.partial(_gather_add_body, offset),
        out_type=jax.ShapeDtypeStruct((E_HALF, F), jnp.float32),
        mesh=mesh,
        compiler_params=pltpu.CompilerParams(needs_layout_passes=False),
        scratch_types=[
            pltpu.VMEM((EPW,), jnp.int32),
            pltpu.VMEM((EPW,), jnp.int32),
            pltpu.VMEM((2, CH_B, F), jnp.float32),
            pltpu.VMEM((2, CH_B, F), jnp.float32),
            pltpu.SemaphoreType.DMA,
            pltpu.SemaphoreType.DMA,
            pltpu.SemaphoreType.DMA,
            pltpu.SemaphoreType.DMA,
            pltpu.SemaphoreType.DMA,
        ],
    )
    return fn(cd, cs, src, dst)


def _edge_mlp_body(g_ref, xl_ref, w1_ref, b1_ref, w2_ref, b2_ref, mt_ref):
    t = lax.dot_general(xl_ref[...], w1_ref[...], _DN_CONTRACT_MINOR,
                        preferred_element_type=jnp.float32)
    h = jnp.maximum(g_ref[...] + t + b1_ref[...], 0.0)
    mt = lax.dot_general(w2_ref[...], h, _DN_CONTRACT_MINOR,
                         preferred_element_type=jnp.float32)
    mt_ref[...] = (mt + b2_ref[...]).reshape(FG, RPG, mt.shape[-1])


def _edge_mlp(g, x_label, w1, b1, w2, b2, offset):
    eb = 1280
    grid = (E_HALF // eb,)
    off_b = offset // eb
    return pl.pallas_call(
        _edge_mlp_body,
        grid=grid,
        in_specs=[
            pl.BlockSpec((eb, F), lambda i: (i, 0)),
            pl.BlockSpec((eb, F), lambda i: (off_b + i, 0)),
            pl.BlockSpec((F, F), lambda i: (0, 0)),
            pl.BlockSpec((1, F), lambda i: (0, 0)),
            pl.BlockSpec((F, F), lambda i: (0, 0)),
            pl.BlockSpec((F, 1), lambda i: (0, 0)),
        ],
        out_specs=pl.BlockSpec((FG, RPG, eb), lambda i: (0, 0, i)),
        out_shape=jax.ShapeDtypeStruct((FG, RPG, E_HALF), jnp.float32),
    )(g, x_label, w1, b1, w2, b2)


def _segmax_body(offset, mt_hbm, dst_hbm, out_hbm, didx_v, mrow_v, acc_v,
                 probe_v, semd0, semd1):
    semd = (semd0, semd1)
    wid = lax.axis_index("s") * NC + lax.axis_index("c")
    fg = wid % FG
    half = wid // FG
    neg_inf = jnp.float32(float("-inf"))
    iota = lax.iota(jnp.int32, L)
    rconst = [jnp.full((L,), r, jnp.int32) for r in range(RPG)]

    def init(i, c):
        for r in range(RPG):
            acc_v[r, pl.ds(i * L, L)] = jnp.full((L,), neg_inf, jnp.float32)
        return c

    lax.fori_loop(0, N_NODES // L, init, 0)

    def start_dma(k, b):
        base = (2 * k + half) * CH_D
        pltpu.async_copy(dst_hbm.at[pl.ds(offset + base, CH_D)], didx_v.at[b],
                         semd[b])
        pltpu.async_copy(mt_hbm.at[fg, :, pl.ds(base, CH_D)], mrow_v.at[b],
                         semd[b])

    def wait_dma(k, b):
        base = (2 * k + half) * CH_D
        pltpu.make_async_copy(dst_hbm.at[pl.ds(offset + base, CH_D)],
                              didx_v.at[b], semd[b]).wait()
        pltpu.make_async_copy(mt_hbm.at[fg, :, pl.ds(base, CH_D)],
                              mrow_v.at[b], semd[b]).wait()

    def update(dstv, vals, mask):
        plsc.store_scatter(probe_v, [dstv], iota, mask=mask)
        got = plsc.load_gather(probe_v, [dstv])
        win = mask & (got == iota)
        curs = [plsc.load_gather(acc_v, [rconst[r], dstv])
                for r in range(RPG)]
        news = [jnp.maximum(curs[r], vals[r]) for r in range(RPG)]
        for r in range(RPG):
            plsc.store_scatter(acc_v, [rconst[r], dstv], news[r], mask=win)
        return mask & (~win)

    def process(k, b, last):
        b2 = 1 - b
        wait_dma(k, b)
        if not last:
            start_dma(k + 1, b2)

        ones = jnp.ones((L,), jnp.bool_)

        def cond(p):
            return jnp.max(p.astype(jnp.int32)) > 0

        def vec(i, c2):
            sa = pl.ds((2 * i) * L, L)
            sb = pl.ds((2 * i + 1) * L, L)
            dsta = didx_v[b, sa]
            dstb = didx_v[b, sb]
            valsa = [mrow_v[b, r, sa] for r in range(RPG)]
            valsb = [mrow_v[b, r, sb] for r in range(RPG)]
            pa = update(dsta, valsa, ones)
            pb = update(dstb, valsb, ones)

            @pl.when(jnp.max((pa | pb).astype(jnp.int32)) > 0)
            def _slow():
                lax.while_loop(cond, lambda p: update(dsta, valsa, p), pa)
                lax.while_loop(cond, lambda p: update(dstb, valsb, p), pb)

            return c2

        lax.fori_loop(0, CH_D // (2 * L), vec, 0)

    start_dma(0, 0)
    process(0, 0, False)

    def pair(ci, carry):
        k = 1 + 2 * ci
        process(k, 1, False)
        process(k + 1, 0, False)
        return carry

    lax.fori_loop(0, 61, pair, 0)
    process(123, 1, False)
    process(124, 0, True)
    pltpu.sync_copy(acc_v, out_hbm.at[half, pl.ds(fg * RPG, RPG), :])


def _segment_max(mt, dst, offset):
    mesh = plsc.VectorSubcoreMesh(
        core_axis_name="c", subcore_axis_name="s",
        num_cores=NC, num_subcores=NS)
    fn = pl.kernel(
        functools.partial(_segmax_body, offset),
        out_type=jax.ShapeDtypeStruct((2, F, N_NODES), jnp.float32),
        mesh=mesh,
        compiler_params=pltpu.CompilerParams(needs_layout_passes=False),
        scratch_types=[
            pltpu.VMEM((2, CH_D), jnp.int32),
            pltpu.VMEM((2, RPG, CH_D), jnp.float32),
            pltpu.VMEM((RPG, N_NODES), jnp.float32),
            pltpu.VMEM((N_NODES,), jnp.int32),
            pltpu.SemaphoreType.DMA,
            pltpu.SemaphoreType.DMA,
        ],
    )
    return fn(mt, dst)


def _merge_body(p1_ref, p2_ref, out_ref):
    neg_inf = jnp.float32(float("-inf"))
    mx = jnp.maximum(jnp.maximum(p1_ref[0], p1_ref[1]),
                     jnp.maximum(p2_ref[0], p2_ref[1]))
    out_ref[...] = jnp.where(mx == neg_inf, jnp.float32(0.0), mx)


def _merge_halves(p1, p2):
    return pl.pallas_call(
        _merge_body,
        grid=(1,),
        in_specs=[
            pl.BlockSpec((2, F, N_NODES), lambda i: (0, 0, 0)),
            pl.BlockSpec((2, F, N_NODES), lambda i: (0, 0, 0)),
        ],
        out_specs=pl.BlockSpec((F, N_NODES), lambda i: (0, 0)),
        out_shape=jax.ShapeDtypeStruct((F, N_NODES), jnp.float32),
    )(p1, p2)


def kernel(x, edge_index, x_label, W_label, W1, b1, W2, b2):
    src = edge_index[0]
    dst = edge_index[1]
    b1r = b1.reshape(1, F)
    b2c = b2.reshape(F, 1)
    cd, cs = _node_tables(x, W_label, W1)
    g1 = _gather_add(cd, cs, src, dst, 0)
    g2 = _gather_add(cd, cs, src, dst, E_HALF)
    mt1 = _edge_mlp(g1, x_label, W1, b1r, W2, b2c, 0)
    mt2 = _edge_mlp(g2, x_label, W1, b1r, W2, b2c, E_HALF)
    p1 = _segment_max(mt1, dst, 0)
    p2 = _segment_max(mt2, dst, E_HALF)
    outt = _merge_halves(p1, p2)
    return outt.T


# speedup vs baseline: 1.8888x; 1.0004x over previous
import functools

import jax
import jax.numpy as jnp
from jax import lax
from jax.experimental import pallas as pl
from jax.experimental.pallas import tpu as pltpu
from jax.experimental.pallas import tpu_sc as plsc

N_NODES = 10000
N_EDGES = 320000
F = 128

NC = 2
NS = 16
L = 16
NW = NC * NS
E_HALF = N_EDGES // 2
EPW = E_HALF // NW
CH_B = 200
NCH_B = EPW // CH_B
CH_D = 640
N_CH_D = E_HALF // CH_D
FG = 16
RPG = F // FG

_DN_CONTRACT_MINOR = (((1,), (1,)), ((), ()))


def _node_tables_body(x_ref, wl_ref, w1_ref, cd_ref, cs_ref):
    x = x_ref[...]
    wl = wl_ref[...]
    a = wl[:, :F]
    b = wl[:, F:]
    w1 = w1_ref[...]
    cd0 = lax.dot_general(x, a - b, _DN_CONTRACT_MINOR,
                          preferred_element_type=jnp.float32)
    cs0 = lax.dot_general(x, b, _DN_CONTRACT_MINOR,
                          preferred_element_type=jnp.float32)
    cd_ref[...] = lax.dot_general(cd0, w1, _DN_CONTRACT_MINOR,
                                  preferred_element_type=jnp.float32)
    cs_ref[...] = lax.dot_general(cs0, w1, _DN_CONTRACT_MINOR,
                                  preferred_element_type=jnp.float32)


def _node_tables(x, w_label, w1):
    nb = 2000
    grid = (N_NODES // nb,)
    return pl.pallas_call(
        _node_tables_body,
        grid=grid,
        in_specs=[
            pl.BlockSpec((nb, F), lambda i: (i, 0)),
            pl.BlockSpec((F, 2 * F), lambda i: (0, 0)),
            pl.BlockSpec((F, F), lambda i: (0, 0)),
        ],
        out_specs=[
            pl.BlockSpec((nb, F), lambda i: (i, 0)),
            pl.BlockSpec((nb, F), lambda i: (i, 0)),
        ],
        out_shape=[
            jax.ShapeDtypeStruct((N_NODES, F), jnp.float32),
            jax.ShapeDtypeStruct((N_NODES, F), jnp.float32),
        ],
    )(x, w_label, w1)


def _gather_add_body(offset, cd_hbm, cs_hbm, src_hbm, dst_hbm, g_hbm,
                     didx_v, sidx_v, cdr_v, csr_v,
                     semi, semg0, semg1, semw0, semw1):
    semg = (semg0, semg1)
    semw = (semw0, semw1)
    wid = lax.axis_index("s") * NC + lax.axis_index("c")
    base_w = offset + wid * EPW
    out_w = wid * EPW

    cpi0 = pltpu.async_copy(dst_hbm.at[pl.ds(base_w, EPW)], didx_v, semi)
    cpi1 = pltpu.async_copy(src_hbm.at[pl.ds(base_w, EPW)], sidx_v, semi)
    cpi0.wait()
    cpi1.wait()

    def start_gather(c, b):
        sl = pl.ds(c * CH_B, CH_B)
        pltpu.async_copy(cd_hbm.at[didx_v.at[sl]], cdr_v.at[b], semg[b])
        pltpu.async_copy(cs_hbm.at[sidx_v.at[sl]], csr_v.at[b], semg[b])

    def wait_gather(c, b):
        sl = pl.ds(c * CH_B, CH_B)
        pltpu.make_async_copy(cd_hbm.at[didx_v.at[sl]], cdr_v.at[b],
                              semg[b]).wait()
        pltpu.make_async_copy(cs_hbm.at[sidx_v.at[sl]], csr_v.at[b],
                              semg[b]).wait()

    def wait_write(c, b):
        pltpu.make_async_copy(cdr_v.at[b],
                              g_hbm.at[pl.ds(out_w + c * CH_B, CH_B), :],
                              semw[b]).wait()

    start_gather(0, 0)

    def process(c, b, first, last):
        b2 = 1 - b
        wait_gather(c, b)
        if not first:
            wait_write(c - 1, b2)
        if not last:
            start_gather(c + 1, b2)

        def row(e, c2):
            for j in range(F // L):
                s = pl.ds(j * L, L)
                cdr_v[b, e, s] = cdr_v[b, e, s] + csr_v[b, e, s]
            return c2

        lax.fori_loop(0, CH_B, row, 0)
        pltpu.async_copy(cdr_v.at[b],
                         g_hbm.at[pl.ds(out_w + c * CH_B, CH_B), :], semw[b])

    process(0, 0, True, False)

    def pair(ci, carry):
        c = 1 + 2 * ci
        process(c, 1, False, False)
        process(c + 1, 0, False, False)
        return carry

    np_ = (NCH_B - 2) // 2
    lax.fori_loop(0, np_, pair, 0)
    for m in range(1 + 2 * np_, NCH_B):
        process(m, m % 2, False, m == NCH_B - 1)
    wait_write(NCH_B - 1, (NCH_B - 1) % 2)


def _gather_add(cd, cs, src, dst, offset):
    mesh = plsc.VectorSubcoreMesh(
        core_axis_name="c", subcore_axis_name="s",
        num_cores=NC, num_subcores=NS)
    fn = pl.kernel(
        functools.partial(_gather_add_body, offset),
        out_type=jax.ShapeDtypeStruct((E_HALF, F), jnp.float32),
        mesh=mesh,
        compiler_params=pltpu.CompilerParams(needs_layout_passes=False),
        scratch_types=[
            pltpu.VMEM((EPW,), jnp.int32),
            pltpu.VMEM((EPW,), jnp.int32),
            pltpu.VMEM((2, CH_B, F), jnp.float32),
            pltpu.VMEM((2, CH_B, F), jnp.float32),
            pltpu.SemaphoreType.DMA,
            pltpu.SemaphoreType.DMA,
            pltpu.SemaphoreType.DMA,
            pltpu.SemaphoreType.DMA,
            pltpu.SemaphoreType.DMA,
        ],
    )
    return fn(cd, cs, src, dst)


def _edge_mlp_body(g_ref, xl_ref, w1_ref, b1_ref, w2_ref, b2_ref, mt_ref):
    t = lax.dot_general(xl_ref[...], w1_ref[...], _DN_CONTRACT_MINOR,
                        preferred_element_type=jnp.float32)
    h = jnp.maximum(g_ref[...] + t + b1_ref[...], 0.0)
    mt = lax.dot_general(w2_ref[...], h, _DN_CONTRACT_MINOR,
                         preferred_element_type=jnp.float32)
    mt_ref[...] = (mt + b2_ref[...]).reshape(FG, RPG, mt.shape[-1])


def _edge_mlp(g, x_label, w1, b1, w2, b2, offset):
    eb = 1280
    grid = (E_HALF // eb,)
    off_b = offset // eb
    return pl.pallas_call(
        _edge_mlp_body,
        grid=grid,
        in_specs=[
            pl.BlockSpec((eb, F), lambda i: (i, 0)),
            pl.BlockSpec((eb, F), lambda i: (off_b + i, 0)),
            pl.BlockSpec((F, F), lambda i: (0, 0)),
            pl.BlockSpec((1, F), lambda i: (0, 0)),
            pl.BlockSpec((F, F), lambda i: (0, 0)),
            pl.BlockSpec((F, 1), lambda i: (0, 0)),
        ],
        out_specs=pl.BlockSpec((FG, RPG, eb), lambda i: (0, 0, i)),
        out_shape=jax.ShapeDtypeStruct((FG, RPG, E_HALF), jnp.float32),
    )(g, x_label, w1, b1, w2, b2)


def _segmax_body(offset, mt_hbm, dst_hbm, out_hbm, didx_v, mrow_v, acc_v,
                 probe_v, semd0, semd1):
    semd = (semd0, semd1)
    wid = lax.axis_index("s") * NC + lax.axis_index("c")
    fg = wid % FG
    half = wid // FG
    neg_inf = jnp.float32(float("-inf"))
    iota = lax.iota(jnp.int32, L)
    rconst = [jnp.full((L,), r, jnp.int32) for r in range(RPG)]

    def init(i, c):
        for r in range(RPG):
            acc_v[r, pl.ds(i * L, L)] = jnp.full((L,), neg_inf, jnp.float32)
        return c

    lax.fori_loop(0, N_NODES // L, init, 0)

    def start_dma(k, b):
        base = (2 * k + half) * CH_D
        pltpu.async_copy(dst_hbm.at[pl.ds(offset + base, CH_D)], didx_v.at[b],
                         semd[b])
        pltpu.async_copy(mt_hbm.at[fg, :, pl.ds(base, CH_D)], mrow_v.at[b],
                         semd[b])

    def wait_dma(k, b):
        base = (2 * k + half) * CH_D
        pltpu.make_async_copy(dst_hbm.at[pl.ds(offset + base, CH_D)],
                              didx_v.at[b], semd[b]).wait()
        pltpu.make_async_copy(mt_hbm.at[fg, :, pl.ds(base, CH_D)],
                              mrow_v.at[b], semd[b]).wait()

    def update(dstv, vals, mask):
        plsc.store_scatter(probe_v, [dstv], iota, mask=mask)
        got = plsc.load_gather(probe_v, [dstv])
        win = mask & (got == iota)
        curs = [plsc.load_gather(acc_v, [rconst[r], dstv])
                for r in range(RPG)]
        news = [jnp.maximum(curs[r], vals[r]) for r in range(RPG)]
        for r in range(RPG):
            plsc.store_scatter(acc_v, [rconst[r], dstv], news[r], mask=win)
        return mask & (~win)

    def process(k, b, last):
        b2 = 1 - b
        wait_dma(k, b)
        if not last:
            start_dma(k + 1, b2)

        ones = jnp.ones((L,), jnp.bool_)

        def cond(p):
            return jnp.max(p.astype(jnp.int32)) > 0

        nv = 4

        def vec(i, c2):
            ss = [pl.ds((nv * i + j) * L, L) for j in range(nv)]
            dsts = [didx_v[b, s] for s in ss]
            vals = [[mrow_v[b, r, s] for r in range(RPG)] for s in ss]
            pends = [update(dsts[j], vals[j], ones) for j in range(nv)]
            anyp = pends[0]
            for j in range(1, nv):
                anyp = anyp | pends[j]

            @pl.when(jnp.max(anyp.astype(jnp.int32)) > 0)
            def _slow():
                for j in range(nv):
                    lax.while_loop(cond,
                                   functools.partial(update, dsts[j], vals[j]),
                                   pends[j])

            return c2

        lax.fori_loop(0, CH_D // (nv * L), vec, 0)

    start_dma(0, 0)
    process(0, 0, False)

    def pair(ci, carry):
        k = 1 + 2 * ci
        process(k, 1, False)
        process(k + 1, 0, False)
        return carry

    lax.fori_loop(0, 61, pair, 0)
    process(123, 1, False)
    process(124, 0, True)
    pltpu.sync_copy(acc_v, out_hbm.at[half, pl.ds(fg * RPG, RPG), :])


def _segment_max(mt, dst, offset):
    mesh = plsc.VectorSubcoreMesh(
        core_axis_name="c", subcore_axis_name="s",
        num_cores=NC, num_subcores=NS)
    fn = pl.kernel(
        functools.partial(_segmax_body, offset),
        out_type=jax.ShapeDtypeStruct((2, F, N_NODES), jnp.float32),
        mesh=mesh,
        compiler_params=pltpu.CompilerParams(needs_layout_passes=False),
        scratch_types=[
            pltpu.VMEM((2, CH_D), jnp.int32),
            pltpu.VMEM((2, RPG, CH_D), jnp.float32),
            pltpu.VMEM((RPG, N_NODES), jnp.float32),
            pltpu.VMEM((N_NODES,), jnp.int32),
            pltpu.SemaphoreType.DMA,
            pltpu.SemaphoreType.DMA,
        ],
    )
    return fn(mt, dst)


def _merge_body(p1_ref, p2_ref, out_ref):
    neg_inf = jnp.float32(float("-inf"))
    mx = jnp.maximum(jnp.maximum(p1_ref[0], p1_ref[1]),
                     jnp.maximum(p2_ref[0], p2_ref[1]))
    out_ref[...] = jnp.where(mx == neg_inf, jnp.float32(0.0), mx)


def _merge_halves(p1, p2):
    return pl.pallas_call(
        _merge_body,
        grid=(1,),
        in_specs=[
            pl.BlockSpec((2, F, N_NODES), lambda i: (0, 0, 0)),
            pl.BlockSpec((2, F, N_NODES), lambda i: (0, 0, 0)),
        ],
        out_specs=pl.BlockSpec((F, N_NODES), lambda i: (0, 0)),
        out_shape=jax.ShapeDtypeStruct((F, N_NODES), jnp.float32),
    )(p1, p2)


def kernel(x, edge_index, x_label, W_label, W1, b1, W2, b2):
    src = edge_index[0]
    dst = edge_index[1]
    b1r = b1.reshape(1, F)
    b2c = b2.reshape(F, 1)
    cd, cs = _node_tables(x, W_label, W1)
    g1 = _gather_add(cd, cs, src, dst, 0)
    g2 = _gather_add(cd, cs, src, dst, E_HALF)
    mt1 = _edge_mlp(g1, x_label, W1, b1r, W2, b2c, 0)
    mt2 = _edge_mlp(g2, x_label, W1, b1r, W2, b2c, E_HALF)
    p1 = _segment_max(mt1, dst, 0)
    p2 = _segment_max(mt2, dst, E_HALF)
    outt = _merge_halves(p1, p2)
    return outt.T
